# baseline (speedup 1.0000x reference)
import jax
import jax.numpy as jnp
from jax.experimental import pallas as pl
from jax.experimental.pallas import tpu as pltpu

_NUM_EXPERTS = 64
_TOP_K = 2
_HIDDEN = 1024
_INTER = 512
_TOKENS = 512
_NPAIR = _TOKENS * _TOP_K

_HI = jax.lax.Precision.HIGHEST

_B = 64
_G = 80


def _routing_body(logits_ref, gidx_ref, wsort_ref, be_ref):
    logits = logits_ref[...]
    lane = jax.lax.broadcasted_iota(jnp.int32, logits.shape, 1)
    big = jnp.int32(10 ** 9)
    m1 = jnp.max(logits, axis=-1, keepdims=True)
    idx1 = jnp.min(jnp.where(logits == m1, lane, big), axis=-1, keepdims=True)
    masked = jnp.where(lane == idx1, -jnp.inf, logits)
    m2 = jnp.max(masked, axis=-1, keepdims=True)
    idx2 = jnp.min(jnp.where(masked == m2, lane, big), axis=-1, keepdims=True)
    w1 = 1.0 / (1.0 + jnp.exp(m2 - m1))
    w2w = 1.0 - w1

    eid = jnp.concatenate([idx1, idx2], axis=0)
    wflat = jnp.concatenate([w1, w2w], axis=0)
    tokf = jnp.concatenate([
        jax.lax.broadcasted_iota(jnp.int32, (_TOKENS, 1), 0).astype(
            jnp.float32)] * 2, axis=0)

    elane = jax.lax.broadcasted_iota(jnp.int32, (_NPAIR, _NUM_EXPERTS), 1)
    onehot = (eid == elane).astype(jnp.float32)

    cnt = jnp.sum(onehot, axis=0, keepdims=True)
    nblk = jnp.floor((cnt + (_B - 1)) * (1.0 / _B))
    ei = jax.lax.broadcasted_iota(jnp.int32, (_NUM_EXPERTS, _NUM_EXPERTS), 0)
    ej = jax.lax.broadcasted_iota(jnp.int32, (_NUM_EXPERTS, _NUM_EXPERTS), 1)
    ut = (ei < ej).astype(jnp.float32)
    pblk = jnp.round(jax.lax.dot_general(
        nblk, ut, (((1,), (0,)), ((), ())), precision=_HI,
        preferred_element_type=jnp.float32))

    ii = jax.lax.broadcasted_iota(jnp.int32, (_NPAIR, _NPAIR), 0)
    jj = jax.lax.broadcasted_iota(jnp.int32, (_NPAIR, _NPAIR), 1)
    ls = (ii > jj).astype(jnp.float32)
    rankmat = jax.lax.dot_general(ls, onehot, (((1,), (0,)), ((), ())),
                                  precision=_HI,
                                  preferred_element_type=jnp.float32)
    rank = jnp.round(jnp.sum(rankmat * onehot, axis=1, keepdims=True))

    pstart = jnp.sum(onehot * pblk, axis=1, keepdims=True)
    ppos = _B * pstart + rank
    gi = jnp.floor(ppos * (1.0 / _B))
    ri = ppos - _B * gi

    gg = jax.lax.broadcasted_iota(jnp.int32, (_G, _NUM_EXPERTS), 0).astype(
        jnp.float32)
    eexp = jax.lax.broadcasted_iota(jnp.int32, (_G, _NUM_EXPERTS), 1).astype(
        jnp.float32)
    active = jnp.where((gg >= pblk) & (gg < pblk + nblk), 1.0, 0.0)
    be_ref[...] = jnp.sum(active * eexp, axis=1, keepdims=True) + jnp.zeros(
        (_G, 128), jnp.float32)

    rr = jax.lax.broadcasted_iota(jnp.int32, (_NPAIR, _B), 1).astype(
        jnp.float32)
    ug = (ri == rr).astype(jnp.float32)
    gcols = jax.lax.broadcasted_iota(jnp.int32, (_NPAIR, _G), 1).astype(
        jnp.float32)
    vg = (gi == gcols).astype(jnp.float32)
    gidx_ref[...] = jnp.round(jax.lax.dot_general(
        ug * (tokf + 1.0), vg, (((0,), (0,)), ((), ())), precision=_HI,
        preferred_element_type=jnp.float32)) - 1.0
    wsort_ref[...] = jax.lax.dot_general(
        ug * wflat, vg, (((0,), (0,)), ((), ())), precision=_HI,
        preferred_element_type=jnp.float32)


def _grouped_body(be_sref, x_ref, gidx_ref, wsort_ref, w13_ref, w2_ref,
                  out_ref):
    g = pl.program_id(0)

    onehot_g = (jax.lax.broadcasted_iota(jnp.int32, (_G, 1), 0) == g
                ).astype(jnp.float32)
    tok_col = jnp.round(jax.lax.dot_general(
        gidx_ref[...], onehot_g, (((1,), (0,)), ((), ())), precision=_HI,
        preferred_element_type=jnp.float32))
    w_col = jax.lax.dot_general(wsort_ref[...], onehot_g,
                                (((1,), (0,)), ((), ())), precision=_HI,
                                preferred_element_type=jnp.float32)

    tlane = jax.lax.broadcasted_iota(jnp.int32, (_B, _TOKENS), 1).astype(
        jnp.float32)
    sel = (tok_col == tlane).astype(jnp.float32)

    x_blk = jax.lax.dot_general(sel, x_ref[...], (((1,), (0,)), ((), ())),
                                preferred_element_type=jnp.float32)
    h = jax.lax.dot_general(x_blk, w13_ref[0], (((1,), (1,)), ((), ())),
                            preferred_element_type=jnp.float32)
    gate = h[:, :_INTER]
    up = h[:, _INTER:]
    act = gate * jax.nn.sigmoid(gate) * up
    o_blk = jax.lax.dot_general(act, w2_ref[0], (((1,), (1,)), ((), ())),
                                preferred_element_type=jnp.float32)

    @pl.when(g == 0)
    def _():
        out_ref[...] = jnp.zeros_like(out_ref)

    out_ref[...] += jax.lax.dot_general(
        sel * w_col, o_blk, (((0,), (0,)), ((), ())),
        preferred_element_type=jnp.float32)


@jax.jit
def kernel(hidden_states, router_logits, w13_weight, w2_weight):
    gidx_f, wsort_f, be_f = pl.pallas_call(
        _routing_body,
        in_specs=[pl.BlockSpec((_TOKENS, _NUM_EXPERTS), lambda: (0, 0))],
        out_specs=[
            pl.BlockSpec((_B, _G), lambda: (0, 0)),
            pl.BlockSpec((_B, _G), lambda: (0, 0)),
            pl.BlockSpec((_G, 128), lambda: (0, 0)),
        ],
        out_shape=[
            jax.ShapeDtypeStruct((_B, _G), jnp.float32),
            jax.ShapeDtypeStruct((_B, _G), jnp.float32),
            jax.ShapeDtypeStruct((_G, 128), jnp.float32),
        ],
    )(router_logits)

    block_expert = be_f[:, 0].astype(jnp.int32)

    out = pl.pallas_call(
        _grouped_body,
        grid_spec=pltpu.PrefetchScalarGridSpec(
            num_scalar_prefetch=1,
            grid=(_G,),
            in_specs=[
                pl.BlockSpec((_TOKENS, _HIDDEN), lambda g, be: (0, 0)),
                pl.BlockSpec((_B, _G), lambda g, be: (0, 0)),
                pl.BlockSpec((_B, _G), lambda g, be: (0, 0)),
                pl.BlockSpec((1, 2 * _INTER, _HIDDEN),
                             lambda g, be: (be[g], 0, 0)),
                pl.BlockSpec((1, _HIDDEN, _INTER),
                             lambda g, be: (be[g], 0, 0)),
            ],
            out_specs=pl.BlockSpec((_TOKENS, _HIDDEN), lambda g, be: (0, 0)),
        ),
        out_shape=jax.ShapeDtypeStruct((_TOKENS, _HIDDEN), jnp.float32),
    )(block_expert, hidden_states, gidx_f, wsort_f, w13_weight, w2_weight)
    return out


# speedup vs baseline: 1.1972x; 1.1972x over previous
import jax
import jax.numpy as jnp
from jax.experimental import pallas as pl

_NUM_EXPERTS = 64
_TOP_K = 2
_HIDDEN = 1024
_INTER = 512
_TOKENS = 512


def _moe_dense_body(logits_ref, x_ref, w13_ref, w2_ref, out_ref):
    e = pl.program_id(0)

    logits = logits_ref[...]
    m1 = jnp.max(logits, axis=-1, keepdims=True)
    lane = jax.lax.broadcasted_iota(jnp.int32, logits.shape, 1)
    big = jnp.int32(10 ** 9)
    idx1 = jnp.min(jnp.where(logits == m1, lane, big), axis=-1, keepdims=True)
    masked = jnp.where(lane == idx1, -jnp.inf, logits)
    m2 = jnp.max(masked, axis=-1, keepdims=True)
    idx2 = jnp.min(jnp.where(masked == m2, lane, big), axis=-1, keepdims=True)
    w1 = 1.0 / (1.0 + jnp.exp(m2 - m1))
    w2w = 1.0 - w1
    col = jnp.where(idx1 == e, w1, 0.0) + jnp.where(idx2 == e, w2w, 0.0)

    x = x_ref[...].astype(jnp.bfloat16)
    w13 = w13_ref[0].astype(jnp.bfloat16)
    h = jax.lax.dot_general(x, w13, (((1,), (1,)), ((), ())),
                            preferred_element_type=jnp.float32)
    gate = h[:, :_INTER]
    up = h[:, _INTER:]
    act = gate * jax.nn.sigmoid(gate) * up
    w2 = w2_ref[0].astype(jnp.bfloat16)
    o = jax.lax.dot_general(act.astype(jnp.bfloat16), w2,
                            (((1,), (1,)), ((), ())),
                            preferred_element_type=jnp.float32)

    @pl.when(e == 0)
    def _():
        out_ref[...] = jnp.zeros_like(out_ref)

    out_ref[...] += col * o


@jax.jit
def kernel(hidden_states, router_logits, w13_weight, w2_weight):
    return pl.pallas_call(
        _moe_dense_body,
        grid=(_NUM_EXPERTS,),
        in_specs=[
            pl.BlockSpec((_TOKENS, _NUM_EXPERTS), lambda e: (0, 0)),
            pl.BlockSpec((_TOKENS, _HIDDEN), lambda e: (0, 0)),
            pl.BlockSpec((1, 2 * _INTER, _HIDDEN), lambda e: (e, 0, 0)),
            pl.BlockSpec((1, _HIDDEN, _INTER), lambda e: (e, 0, 0)),
        ],
        out_specs=pl.BlockSpec((_TOKENS, _HIDDEN), lambda e: (0, 0)),
        out_shape=jax.ShapeDtypeStruct((_TOKENS, _HIDDEN), jnp.float32),
    )(router_logits, hidden_states, w13_weight, w2_weight)
